# baseline (speedup 1.0000x reference)
import functools

import jax
import jax.numpy as jnp
from jax.experimental import pallas as pl
from jax.experimental.pallas import tpu as pltpu


def _chamfer_body(a_ref, bt_ref, s1_ref, s2_ref, *, n1, n2, tm):
    bt = bt_ref[0]
    sq2 = jnp.sum(bt * bt, axis=0, keepdims=True)

    def body(i, carry):
        s1, d2 = carry
        atile = a_ref[0, pl.ds(i * tm, tm), :]
        inner = jnp.dot(atile, bt, preferred_element_type=jnp.float32)
        sq1 = jnp.sum(atile * atile, axis=1, keepdims=True)
        pair = jnp.maximum(sq1 + sq2 - 2.0 * inner, 0.0)
        s1 = s1 + jnp.sum(jnp.min(pair, axis=1))
        d2 = jnp.minimum(d2, jnp.min(pair, axis=0, keepdims=True))
        return s1, d2

    s1 = jnp.float32(0.0)
    d2 = jnp.full((1, n2), jnp.inf, dtype=jnp.float32)
    s1, d2 = jax.lax.fori_loop(0, n1 // tm, body, (s1, d2))
    s1_ref[0] = jnp.full((1, 128), s1, dtype=jnp.float32)
    s2_ref[0] = jnp.full((1, 128), jnp.sum(d2), dtype=jnp.float32)


def kernel(xyz1, xyz2):
    b, n1, d = xyz1.shape
    _, n2, _ = xyz2.shape
    tm = 256
    c2t = xyz2.transpose(0, 2, 1)

    s1, s2 = pl.pallas_call(
        functools.partial(_chamfer_body, n1=n1, n2=n2, tm=tm),
        grid=(b,),
        in_specs=[
            pl.BlockSpec((1, n1, d), lambda i: (i, 0, 0)),
            pl.BlockSpec((1, d, n2), lambda i: (i, 0, 0)),
        ],
        out_specs=[
            pl.BlockSpec((1, 1, 128), lambda i: (i, 0, 0)),
            pl.BlockSpec((1, 1, 128), lambda i: (i, 0, 0)),
        ],
        out_shape=[
            jax.ShapeDtypeStruct((b, 1, 128), jnp.float32),
            jax.ShapeDtypeStruct((b, 1, 128), jnp.float32),
        ],
        compiler_params=pltpu.CompilerParams(
            dimension_semantics=("parallel",),
        ),
    )(xyz1, c2t)

    return jnp.sum(s1[:, 0, 0]) / (b * n1) + jnp.sum(s2[:, 0, 0]) / (b * n2)


# speedup vs baseline: 1.1376x; 1.1376x over previous
import functools

import jax
import jax.numpy as jnp
from jax.experimental import pallas as pl
from jax.experimental.pallas import tpu as pltpu


def _chamfer_body(a_ref, bt_ref, s1_ref, s2_ref, *, n1, n2, tm):
    bt = bt_ref[0]

    def body(i, carry):
        s1, d2 = carry
        atile = a_ref[0, pl.ds(i * tm, tm), :]
        pair = jnp.dot(atile, bt, preferred_element_type=jnp.float32)
        rowmin = jnp.min(pair, axis=1)
        s1 = s1 + jnp.sum(jnp.maximum(rowmin, 0.0))
        d2 = jnp.minimum(d2, jnp.min(pair, axis=0, keepdims=True))
        return s1, d2

    s1 = jnp.float32(0.0)
    d2 = jnp.full((1, n2), jnp.inf, dtype=jnp.float32)
    s1, d2 = jax.lax.fori_loop(0, n1 // tm, body, (s1, d2))
    s2 = jnp.sum(jnp.maximum(d2, 0.0))
    s1_ref[0] = jnp.full((1, 128), s1, dtype=jnp.float32)
    s2_ref[0] = jnp.full((1, 128), s2, dtype=jnp.float32)


def kernel(xyz1, xyz2):
    b, n1, _ = xyz1.shape
    _, n2, _ = xyz2.shape
    tm = 256

    sq1 = jnp.sum(xyz1 * xyz1, axis=-1, keepdims=True)
    sq2 = jnp.sum(xyz2 * xyz2, axis=-1, keepdims=True)
    ones1 = jnp.ones_like(sq1)
    aug1 = jnp.concatenate([xyz1, sq1, ones1], axis=-1)
    aug2 = jnp.concatenate([-2.0 * xyz2, jnp.ones_like(sq2), sq2], axis=-1)
    aug2t = aug2.transpose(0, 2, 1)

    s1, s2 = pl.pallas_call(
        functools.partial(_chamfer_body, n1=n1, n2=n2, tm=tm),
        grid=(b,),
        in_specs=[
            pl.BlockSpec((1, n1, 5), lambda i: (i, 0, 0)),
            pl.BlockSpec((1, 5, n2), lambda i: (i, 0, 0)),
        ],
        out_specs=[
            pl.BlockSpec((1, 1, 128), lambda i: (i, 0, 0)),
            pl.BlockSpec((1, 1, 128), lambda i: (i, 0, 0)),
        ],
        out_shape=[
            jax.ShapeDtypeStruct((b, 1, 128), jnp.float32),
            jax.ShapeDtypeStruct((b, 1, 128), jnp.float32),
        ],
        compiler_params=pltpu.CompilerParams(
            dimension_semantics=("parallel",),
        ),
    )(aug1, aug2t)

    return jnp.sum(s1[:, 0, 0]) / (b * n1) + jnp.sum(s2[:, 0, 0]) / (b * n2)
